# baseline (speedup 1.0000x reference)
import functools

import jax
import jax.numpy as jnp
from jax import lax
from jax.experimental import pallas as pl
from jax.experimental.pallas import tpu as pltpu
from jax.experimental.pallas import tpu_sc as plsc

NC = 2
NS = 16
NW = NC * NS
L = 16
D = 128
CH = 128


def _pad_workers(x, chunks):
    total = NW * chunks * CH
    pad = total - x.shape[0]
    x = jnp.concatenate([x, jnp.zeros((pad,), x.dtype)])
    return x.reshape(NW, chunks, CH)


def _sc_spmm(pre, s1, d1, w1, s2, d2, w2, c1, c2, n):
    n_pad = ((n + NS * CH - 1) // (NS * CH)) * NS * CH
    rows_t = n_pad // NS
    nz = rows_t // CH
    sb = c1 // 2
    assert c2 <= sb and sb % 8 == 0 and c1 % 4 == 0 and c2 % 2 == 0

    mesh = plsc.VectorSubcoreMesh(core_axis_name="c", subcore_axis_name="s")

    @functools.partial(
        pl.kernel,
        mesh=mesh,
        out_type=[
            jax.ShapeDtypeStruct((NC, n_pad, D), jnp.float32),
            jax.ShapeDtypeStruct((NC, n_pad, D), jnp.float32),
        ],
        scratch_types=[
            pltpu.VMEM((CH, D), jnp.float32),
            pltpu.VMEM((CH, D), jnp.float32),
            pltpu.VMEM((sb, CH), jnp.int32),
            pltpu.VMEM((sb, CH), jnp.int32),
            pltpu.VMEM((sb, CH), jnp.float32),
            pltpu.VMEM_SHARED((n_pad, D), jnp.float32),
            pltpu.SemaphoreType.DMA,
            pltpu.SemaphoreType.DMA,
        ],
    )
    def spmm(pre_h, s1_h, d1_h, w1_h, s2_h, d2_h, w2_h, outp_h, outq_h,
             rows0_v, rows1_v, sidx_v, didx_v, wv_v, acc_s, sem0, sem1):
        c = lax.axis_index("c")
        s = lax.axis_index("s")
        wid = c * NS + s
        base = s * rows_t

        def zero_acc_slice():
            def zr(r, carry):
                for g in range(D // L):
                    rows0_v[r, pl.ds(g * L, L)] = jnp.zeros((L,), jnp.float32)
                return carry
            lax.fori_loop(0, CH, zr, 0)
            for k in range(nz):
                pltpu.sync_copy(rows0_v, acc_s.at[pl.ds(base + k * CH, CH)])

        def mul_scatter(rows_v, j):
            def mulgrp(g, carry2):
                w16 = wv_v[j, pl.ds(g * L, L)]
                for lane in range(L):
                    w = w16[lane]
                    r = g * L + lane
                    for col in range(D // L):
                        sl = pl.ds(col * L, L)
                        rows_v[r, sl] = rows_v[r, sl] * w
                return carry2
            with jax.named_scope("mul"):
                lax.fori_loop(0, CH // L, mulgrp, 0)
            with jax.named_scope("scat"):
                pltpu.sync_copy(rows_v, acc_s.at[didx_v.at[j]], add=True)

        def pair_loop(half):
            def pair(g, carry):
                j0 = 2 * g
                pltpu.async_copy(pre_h.at[sidx_v.at[j0 + 1]], rows1_v, sem1)
                pltpu.make_async_copy(pre_h.at[sidx_v.at[j0]],
                                      rows0_v, sem0).wait()
                mul_scatter(rows0_v, j0)

                @pl.when(g + 1 < half)
                def _():
                    pltpu.async_copy(pre_h.at[sidx_v.at[j0 + 2]],
                                     rows0_v, sem0)
                pltpu.make_async_copy(pre_h.at[sidx_v.at[j0 + 1]],
                                      rows1_v, sem1).wait()
                mul_scatter(rows1_v, j0 + 1)
                return carry
            lax.fori_loop(0, half, pair, 0)

        def phase(s_h, d_h, w_h, nchunks, out_h):
            with jax.named_scope("zero"):
                zero_acc_slice()
                plsc.subcore_barrier()
            for st in range(-(-nchunks // sb)):
                cs = min(sb, nchunks - st * sb)
                pltpu.sync_copy(s_h.at[wid, pl.ds(st * sb, cs)],
                                sidx_v.at[pl.ds(0, cs)])
                pltpu.sync_copy(d_h.at[wid, pl.ds(st * sb, cs)],
                                didx_v.at[pl.ds(0, cs)])
                pltpu.sync_copy(w_h.at[wid, pl.ds(st * sb, cs)],
                                wv_v.at[pl.ds(0, cs)])
                pltpu.async_copy(pre_h.at[sidx_v.at[0]], rows0_v, sem0)
                pair_loop(cs // 2)
            with jax.named_scope("dbar"):
                plsc.subcore_barrier()
            bufs = ((rows0_v, sem0), (rows1_v, sem1))
            for k in range(nz):
                buf, bsem = bufs[k % 2]
                if k >= 2:
                    with jax.named_scope("dwaitw"):
                        pltpu.make_async_copy(
                            buf, out_h.at[c, pl.ds(base + (k - 2) * CH, CH)],
                            bsem).wait()
                with jax.named_scope("dread"):
                    pltpu.sync_copy(acc_s.at[pl.ds(base + k * CH, CH)], buf)
                pltpu.async_copy(
                    buf, out_h.at[c, pl.ds(base + k * CH, CH)], bsem)
            for k in range(max(0, nz - 2), nz):
                buf, bsem = bufs[k % 2]
                with jax.named_scope("dwaitw"):
                    pltpu.make_async_copy(
                        buf, out_h.at[c, pl.ds(base + k * CH, CH)],
                        bsem).wait()

        phase(s1_h, d1_h, w1_h, c1, outp_h)
        phase(s2_h, d2_h, w2_h, c2, outq_h)

    return spmm(pre, s1, d1, w1, s2, d2, w2)


def _tc_dense(pre, P, Q, Qw, W1, W2, bb, Mm, n_users, n):
    R = 400
    nblk = n // R
    ublk = n_users // R
    assert n % R == 0 and n_users % R == 0

    def body(pre_b, p0_b, p1_b, q0_b, q1_b, qw_b, w1_b, w2_b, bb_b, mm_b,
             out_b):
        agg = p0_b[0] + p1_b[0]
        pq = q0_b[0] + q1_b[0]
        hi = jax.lax.Precision.HIGHEST
        t = jnp.tanh(jax.lax.dot(agg, qw_b[0], precision=hi,
                                 preferred_element_type=jnp.float32))
        h = (jax.lax.dot(pre_b[...], w1_b[0], precision=hi,
                         preferred_element_type=jnp.float32)
             + jax.lax.dot(t, w2_b[0], precision=hi,
                           preferred_element_type=jnp.float32)
             + bb_b[0])
        e = jnp.where(h >= 0, h, 0.01 * h)
        nrm = jnp.sqrt(jnp.sum(e * e, axis=1, keepdims=True))
        e = e / (nrm + 1e-12)
        out_b[...] = e + jnp.tanh(jax.lax.dot(pq, mm_b[0], precision=hi,
                                              preferred_element_type=jnp.float32))

    def wmap(i):
        sel = jnp.where(i < ublk, 0, 1)
        return (sel, 0, 0)

    return pl.pallas_call(
        body,
        grid=(nblk,),
        in_specs=[
            pl.BlockSpec((R, D), lambda i: (i, 0)),
            pl.BlockSpec((1, R, D), lambda i: (0, i, 0)),
            pl.BlockSpec((1, R, D), lambda i: (1, i, 0)),
            pl.BlockSpec((1, R, D), lambda i: (0, i, 0)),
            pl.BlockSpec((1, R, D), lambda i: (1, i, 0)),
            pl.BlockSpec((1, D, D), wmap),
            pl.BlockSpec((1, D, D), wmap),
            pl.BlockSpec((1, D, D), wmap),
            pl.BlockSpec((1, 1, D), wmap),
            pl.BlockSpec((1, D, D), wmap),
        ],
        out_specs=pl.BlockSpec((R, D), lambda i: (i, 0)),
        out_shape=jax.ShapeDtypeStruct((n, D), jnp.float32),
    )(pre, P, P, Q, Q, Qw, W1, W2, bb, Mm)


def kernel(edge_index, edge_weight, sym_pair_edge_index, sym_pair_weight,
           herb_pair_edge_index, herb_pair_weight, user_embedding,
           item_embedding, Q_user_0, W_gc_user_0, b_gc_user_0, Q_item_0,
           W_gc_item_0, b_gc_item_0, M_user, M_item):
    n_users = user_embedding.shape[0]
    n_items = item_embedding.shape[0]
    n = n_users + n_items
    pre = jnp.concatenate([user_embedding, item_embedding], axis=0)

    e1 = edge_weight.shape[0]
    c1 = 2 * -(-e1 // (2 * NW * CH))
    s1 = _pad_workers(edge_index[0], c1)
    d1 = _pad_workers(edge_index[1], c1)
    w1 = _pad_workers(edge_weight, c1)

    s2r = jnp.concatenate([sym_pair_edge_index[0],
                           herb_pair_edge_index[0] + n_users])
    d2r = jnp.concatenate([sym_pair_edge_index[1],
                           herb_pair_edge_index[1] + n_users])
    w2r = jnp.concatenate([sym_pair_weight, herb_pair_weight])
    e2 = w2r.shape[0]
    c2 = 2 * -(-e2 // (2 * NW * CH))
    s2 = _pad_workers(s2r, c2)
    d2 = _pad_workers(d2r, c2)
    w2 = _pad_workers(w2r, c2)

    P, Q = _sc_spmm(pre, s1, d1, w1, s2, d2, w2, c1, c2, n)

    Qw = jnp.stack([Q_user_0, Q_item_0])
    W1 = jnp.stack([W_gc_user_0[:D], W_gc_item_0[:D]])
    W2 = jnp.stack([W_gc_user_0[D:], W_gc_item_0[D:]])
    bb = jnp.stack([b_gc_user_0, b_gc_item_0])
    Mm = jnp.stack([M_user, M_item])
    return _tc_dense(pre, P, Q, Qw, W1, W2, bb, Mm, n_users, n)


# speedup vs baseline: 2.3241x; 2.3241x over previous
import functools

import jax
import jax.numpy as jnp
from jax import lax
from jax.experimental import pallas as pl
from jax.experimental.pallas import tpu as pltpu
from jax.experimental.pallas import tpu_sc as plsc

NC = 2
NS = 16
NW = NC * NS
L = 16
D = 128
CH = 128


def _pad_workers(x, chunks, spread=0):
    total = NW * chunks * CH
    pad = total - x.shape[0]
    if spread:
        fill = jnp.arange(pad, dtype=x.dtype) % spread
    else:
        fill = jnp.zeros((pad,), x.dtype)
    x = jnp.concatenate([x, fill])
    return x.reshape(NW, chunks, CH)


def _sc_spmm(pre, s1, d1, w1, s2, d2, w2, c1, c2, n):
    n_pad = ((n + NS * CH - 1) // (NS * CH)) * NS * CH
    rows_t = n_pad // NS
    nz = rows_t // CH
    sb = c1 // 2
    assert c2 <= sb and sb % 8 == 0 and c1 % 4 == 0 and c2 % 2 == 0

    mesh = plsc.VectorSubcoreMesh(core_axis_name="c", subcore_axis_name="s")

    @functools.partial(
        pl.kernel,
        mesh=mesh,
        out_type=[
            jax.ShapeDtypeStruct((NC, n_pad, D), jnp.float32),
            jax.ShapeDtypeStruct((NC, n_pad, D), jnp.float32),
        ],
        scratch_types=[
            pltpu.VMEM((CH, D), jnp.float32),
            pltpu.VMEM((CH, D), jnp.float32),
            pltpu.VMEM((sb, CH), jnp.int32),
            pltpu.VMEM((sb, CH), jnp.int32),
            pltpu.VMEM((sb, CH), jnp.float32),
            pltpu.VMEM_SHARED((n_pad, D), jnp.float32),
            pltpu.SemaphoreType.DMA,
            pltpu.SemaphoreType.DMA,
        ],
    )
    def spmm(pre_h, s1_h, d1_h, w1_h, s2_h, d2_h, w2_h, outp_h, outq_h,
             rows0_v, rows1_v, sidx_v, didx_v, wv_v, acc_s, sem0, sem1):
        c = lax.axis_index("c")
        s = lax.axis_index("s")
        wid = c * NS + s
        base = s * rows_t

        def zero_acc_slice():
            def zr(r, carry):
                for g in range(D // L):
                    rows0_v[r, pl.ds(g * L, L)] = jnp.zeros((L,), jnp.float32)
                return carry
            lax.fori_loop(0, CH, zr, 0)
            for k in range(nz):
                pltpu.sync_copy(rows0_v, acc_s.at[pl.ds(base + k * CH, CH)])

        def mul_scatter(rows_v, j):
            def mulgrp(g, carry2):
                w16 = wv_v[j, pl.ds(g * L, L)]
                for lane in range(L):
                    w = w16[lane]
                    r = g * L + lane
                    for col in range(D // L):
                        sl = pl.ds(col * L, L)
                        rows_v[r, sl] = rows_v[r, sl] * w
                return carry2
            with jax.named_scope("mul"):
                lax.fori_loop(0, CH // L, mulgrp, 0)
            with jax.named_scope("scat"):
                pltpu.sync_copy(rows_v, acc_s.at[didx_v.at[j]], add=True)

        def pair_loop(half):
            def pair(g, carry):
                j0 = 2 * g
                pltpu.async_copy(pre_h.at[sidx_v.at[j0 + 1]], rows1_v, sem1)
                pltpu.make_async_copy(pre_h.at[sidx_v.at[j0]],
                                      rows0_v, sem0).wait()
                mul_scatter(rows0_v, j0)

                @pl.when(g + 1 < half)
                def _():
                    pltpu.async_copy(pre_h.at[sidx_v.at[j0 + 2]],
                                     rows0_v, sem0)
                pltpu.make_async_copy(pre_h.at[sidx_v.at[j0 + 1]],
                                      rows1_v, sem1).wait()
                mul_scatter(rows1_v, j0 + 1)
                return carry
            lax.fori_loop(0, half, pair, 0)

        def phase(s_h, d_h, w_h, nchunks, out_h):
            with jax.named_scope("zero"):
                zero_acc_slice()
                plsc.subcore_barrier()
            for st in range(-(-nchunks // sb)):
                cs = min(sb, nchunks - st * sb)
                pltpu.sync_copy(s_h.at[wid, pl.ds(st * sb, cs)],
                                sidx_v.at[pl.ds(0, cs)])
                pltpu.sync_copy(d_h.at[wid, pl.ds(st * sb, cs)],
                                didx_v.at[pl.ds(0, cs)])
                pltpu.sync_copy(w_h.at[wid, pl.ds(st * sb, cs)],
                                wv_v.at[pl.ds(0, cs)])
                pltpu.async_copy(pre_h.at[sidx_v.at[0]], rows0_v, sem0)
                pair_loop(cs // 2)
            with jax.named_scope("dbar"):
                plsc.subcore_barrier()
            bufs = ((rows0_v, sem0), (rows1_v, sem1))
            for k in range(nz):
                buf, bsem = bufs[k % 2]
                if k >= 2:
                    with jax.named_scope("dwaitw"):
                        pltpu.make_async_copy(
                            buf, out_h.at[c, pl.ds(base + (k - 2) * CH, CH)],
                            bsem).wait()
                with jax.named_scope("dread"):
                    pltpu.sync_copy(acc_s.at[pl.ds(base + k * CH, CH)], buf)
                pltpu.async_copy(
                    buf, out_h.at[c, pl.ds(base + k * CH, CH)], bsem)
            for k in range(max(0, nz - 2), nz):
                buf, bsem = bufs[k % 2]
                with jax.named_scope("dwaitw"):
                    pltpu.make_async_copy(
                        buf, out_h.at[c, pl.ds(base + k * CH, CH)],
                        bsem).wait()

        phase(s1_h, d1_h, w1_h, c1, outp_h)
        phase(s2_h, d2_h, w2_h, c2, outq_h)

    return spmm(pre, s1, d1, w1, s2, d2, w2)


def _tc_dense(pre, P, Q, Qw, W1, W2, bb, Mm, n_users, n):
    R = 400
    nblk = n // R
    ublk = n_users // R
    assert n % R == 0 and n_users % R == 0

    def body(pre_b, p0_b, p1_b, q0_b, q1_b, qw_b, w1_b, w2_b, bb_b, mm_b,
             out_b):
        agg = p0_b[0] + p1_b[0]
        pq = q0_b[0] + q1_b[0]
        hi = jax.lax.Precision.HIGHEST
        t = jnp.tanh(jax.lax.dot(agg, qw_b[0], precision=hi,
                                 preferred_element_type=jnp.float32))
        h = (jax.lax.dot(pre_b[...], w1_b[0], precision=hi,
                         preferred_element_type=jnp.float32)
             + jax.lax.dot(t, w2_b[0], precision=hi,
                           preferred_element_type=jnp.float32)
             + bb_b[0])
        e = jnp.where(h >= 0, h, 0.01 * h)
        nrm = jnp.sqrt(jnp.sum(e * e, axis=1, keepdims=True))
        e = e / (nrm + 1e-12)
        out_b[...] = e + jnp.tanh(jax.lax.dot(pq, mm_b[0], precision=hi,
                                              preferred_element_type=jnp.float32))

    def wmap(i):
        sel = jnp.where(i < ublk, 0, 1)
        return (sel, 0, 0)

    return pl.pallas_call(
        body,
        grid=(nblk,),
        in_specs=[
            pl.BlockSpec((R, D), lambda i: (i, 0)),
            pl.BlockSpec((1, R, D), lambda i: (0, i, 0)),
            pl.BlockSpec((1, R, D), lambda i: (1, i, 0)),
            pl.BlockSpec((1, R, D), lambda i: (0, i, 0)),
            pl.BlockSpec((1, R, D), lambda i: (1, i, 0)),
            pl.BlockSpec((1, D, D), wmap),
            pl.BlockSpec((1, D, D), wmap),
            pl.BlockSpec((1, D, D), wmap),
            pl.BlockSpec((1, 1, D), wmap),
            pl.BlockSpec((1, D, D), wmap),
        ],
        out_specs=pl.BlockSpec((R, D), lambda i: (i, 0)),
        out_shape=jax.ShapeDtypeStruct((n, D), jnp.float32),
    )(pre, P, P, Q, Q, Qw, W1, W2, bb, Mm)


def kernel(edge_index, edge_weight, sym_pair_edge_index, sym_pair_weight,
           herb_pair_edge_index, herb_pair_weight, user_embedding,
           item_embedding, Q_user_0, W_gc_user_0, b_gc_user_0, Q_item_0,
           W_gc_item_0, b_gc_item_0, M_user, M_item):
    n_users = user_embedding.shape[0]
    n_items = item_embedding.shape[0]
    n = n_users + n_items
    pre = jnp.concatenate([user_embedding, item_embedding], axis=0)

    e1 = edge_weight.shape[0]
    c1 = 2 * -(-e1 // (2 * NW * CH))
    s1 = _pad_workers(edge_index[0], c1, spread=n)
    d1 = _pad_workers(edge_index[1], c1, spread=n)
    w1 = _pad_workers(edge_weight, c1)

    s2r = jnp.concatenate([sym_pair_edge_index[0],
                           herb_pair_edge_index[0] + n_users])
    d2r = jnp.concatenate([sym_pair_edge_index[1],
                           herb_pair_edge_index[1] + n_users])
    w2r = jnp.concatenate([sym_pair_weight, herb_pair_weight])
    e2 = w2r.shape[0]
    c2 = 2 * -(-e2 // (2 * NW * CH))
    s2 = _pad_workers(s2r, c2, spread=n)
    d2 = _pad_workers(d2r, c2, spread=n)
    w2 = _pad_workers(w2r, c2)

    P, Q = _sc_spmm(pre, s1, d1, w1, s2, d2, w2, c1, c2, n)

    Qw = jnp.stack([Q_user_0, Q_item_0])
    W1 = jnp.stack([W_gc_user_0[:D], W_gc_item_0[:D]])
    W2 = jnp.stack([W_gc_user_0[D:], W_gc_item_0[D:]])
    bb = jnp.stack([b_gc_user_0, b_gc_item_0])
    Mm = jnp.stack([M_user, M_item])
    return _tc_dense(pre, P, Q, Qw, W1, W2, bb, Mm, n_users, n)
